# baseline (speedup 1.0000x reference)
import functools

import jax
import jax.numpy as jnp
from jax import lax
from jax.experimental import pallas as pl
from jax.experimental.pallas import tpu as pltpu
from jax.experimental.pallas import tpu_sc as plsc


def _make_sc_gather(vocab, dim, batch):
    info = plsc.get_sparse_core_info()
    num_cores, num_subcores = info.num_cores, info.num_subcores
    num_workers = num_cores * num_subcores
    assert batch % (8 * num_workers) == 0
    b_per_w = batch // num_workers
    mesh = plsc.VectorSubcoreMesh(core_axis_name="c", subcore_axis_name="s")

    @functools.partial(
        pl.kernel,
        mesh=mesh,
        out_type=jax.ShapeDtypeStruct((dim, batch), jnp.float32),
        scratch_types=[
            pltpu.VMEM((b_per_w,), jnp.int32),
            pltpu.VMEM((dim, b_per_w), jnp.float32),
            pltpu.SemaphoreType.DMA,
        ],
        compiler_params=pltpu.CompilerParams(use_tc_tiling_on_sc=False),
    )
    def gather_kernel(idx_hbm, table_hbm, out_hbm, idx_v, rows_v, sem):
        wid = lax.axis_index("s") * num_cores + lax.axis_index("c")
        base = wid * b_per_w
        pltpu.sync_copy(idx_hbm.at[pl.ds(base, b_per_w)], idx_v)
        copies = [
            pltpu.async_copy(table_hbm.at[e].at[idx_v], rows_v.at[e], sem)
            for e in range(dim)
        ]
        for c in copies:
            c.wait()
        pltpu.sync_copy(rows_v, out_hbm.at[:, pl.ds(base, b_per_w)])

    return gather_kernel


def kernel(node_idx, table):
    batch = node_idx.shape[0]
    vocab, dim = table.shape
    gather = _make_sc_gather(vocab, dim, batch)
    out_t = gather(node_idx.astype(jnp.int32), jnp.swapaxes(table, 0, 1))
    return jnp.swapaxes(out_t, 0, 1)


# speedup vs baseline: 4.9130x; 4.9130x over previous
import functools

import jax
import jax.numpy as jnp
from jax import lax
from jax.experimental import pallas as pl
from jax.experimental.pallas import tpu as pltpu
from jax.experimental.pallas import tpu_sc as plsc

_LANES = 16


def _make_sc_gather(vocab, dim, batch):
    info = plsc.get_sparse_core_info()
    num_cores, num_subcores = info.num_cores, info.num_subcores
    num_workers = num_cores * num_subcores
    rows_per_pack = 128 // dim
    packed_rows = vocab // rows_per_pack
    assert batch % (8 * num_workers) == 0 and vocab % rows_per_pack == 0
    b_per_w = batch // num_workers
    mesh = plsc.VectorSubcoreMesh(core_axis_name="c", subcore_axis_name="s")

    @functools.partial(
        pl.kernel,
        mesh=mesh,
        out_type=jax.ShapeDtypeStruct((dim, batch), jnp.float32),
        scratch_types=[
            pltpu.VMEM((b_per_w,), jnp.int32),
            pltpu.VMEM((b_per_w,), jnp.int32),
            pltpu.VMEM((b_per_w, 128), jnp.float32),
            pltpu.VMEM((dim, b_per_w), jnp.float32),
            pltpu.SemaphoreType.DMA,
        ],
        compiler_params=pltpu.CompilerParams(needs_layout_passes=False),
    )
    def gather_kernel(idx_hbm, packed_hbm, out_hbm, idx_v, idx4_v, rows_v,
                      out_v, sem):
        wid = lax.axis_index("s") * num_cores + lax.axis_index("c")
        base = wid * b_per_w
        pltpu.sync_copy(idx_hbm.at[pl.ds(base, b_per_w)], idx_v)

        def idx4_body(c, _):
            v = idx_v[pl.ds(c * _LANES, _LANES)]
            idx4_v[pl.ds(c * _LANES, _LANES)] = lax.shift_right_logical(v, 2)
            return _

        lax.fori_loop(0, b_per_w // _LANES, idx4_body, 0, unroll=4)

        pltpu.async_copy(packed_hbm.at[idx4_v], rows_v, sem).wait()

        iota = lax.iota(jnp.int32, _LANES)

        def extract_body(jc, _):
            row_ids = jc * _LANES + iota
            q = lax.bitwise_and(idx_v[pl.ds(jc * _LANES, _LANES)], 3)
            col_base = lax.shift_left(q, 5)
            for e in range(dim):
                out_v[e, pl.ds(jc * _LANES, _LANES)] = plsc.load_gather(
                    rows_v, [row_ids, col_base + e]
                )
            return _

        lax.fori_loop(0, b_per_w // _LANES, extract_body, 0)

        pltpu.sync_copy(out_v, out_hbm.at[:, pl.ds(base, b_per_w)])

    return gather_kernel


def kernel(node_idx, table):
    batch = node_idx.shape[0]
    vocab, dim = table.shape
    gather = _make_sc_gather(vocab, dim, batch)
    packed = jnp.reshape(table, (vocab * dim // 128, 128))
    out_t = gather(node_idx.astype(jnp.int32), packed)
    return jnp.swapaxes(out_t, 0, 1)
